# Initial kernel scaffold; baseline (speedup 1.0000x reference)
import jax
import jax.numpy as jnp
from jax.experimental import pallas as pl


def kernel(x, emb_w):
    raise NotImplementedError("write your pallas kernel here")


# speedup vs baseline: 1.0514x; 1.0514x over previous
import functools

import jax
import jax.numpy as jnp
from jax import lax
from jax.experimental import pallas as pl
from jax.experimental.pallas import tpu as pltpu
from jax.experimental.pallas import tpu_sc as plsc

B, C, L = 32, 256, 576
K = 8192
N = B * L
KC = 1024
NKC = K // KC
NW = 32
ROWS_PER_W = N // NW
CH = 96
NCH = ROWS_PER_W // CH
BETA = 0.25


def _argmin_body(x_ref, emb_ref, idx_ref, loss_ref):
    b = pl.program_id(0)
    xb = x_ref[0]
    xx = jnp.sum(xb * xb, axis=0)

    def step(kc, carry):
        rmin, ridx = carry
        e = emb_ref[pl.ds(kc * KC, KC), :]
        ee = jnp.sum(e * e, axis=1)
        dots = lax.dot_general(
            xb, e, (((0,), (1,)), ((), ())),
            preferred_element_type=jnp.float32,
        )
        d2 = (xx[:, None] + ee[None, :]) - 2.0 * dots
        z = jnp.maximum(d2, 0.0)
        cmin = jnp.min(z, axis=1)
        lane = lax.broadcasted_iota(jnp.int32, (L, KC), 1)
        cidx = jnp.min(jnp.where(z == cmin[:, None], lane, K), axis=1) + kc * KC
        upd = cmin < rmin
        return jnp.where(upd, cmin, rmin), jnp.where(upd, cidx, ridx)

    rmin0 = jnp.full((L,), jnp.inf, jnp.float32)
    ridx0 = jnp.zeros((L,), jnp.int32)
    rmin, ridx = lax.fori_loop(0, NKC, step, (rmin0, ridx0))
    idx_ref[0, 0, :] = ridx

    @pl.when(b == 0)
    def _():
        loss_ref[0, 0] = 0.0

    loss_ref[0, 0] += jnp.sum(rmin)

    @pl.when(b == B - 1)
    def _():
        q = loss_ref[0, 0] / jnp.float32(N * C)
        loss_ref[0, 0] = q + BETA * q


def _argmin_call(x, emb_w):
    return pl.pallas_call(
        _argmin_body,
        grid=(B,),
        in_specs=[
            pl.BlockSpec((1, C, L), lambda b: (b, 0, 0)),
            pl.BlockSpec((K, C), lambda b: (0, 0)),
        ],
        out_specs=[
            pl.BlockSpec((1, 1, L), lambda b: (b, 0, 0)),
            pl.BlockSpec(block_shape=(1, 1), index_map=lambda b: (0, 0),
                         memory_space=pltpu.SMEM),
        ],
        out_shape=[
            jax.ShapeDtypeStruct((B, 1, L), jnp.int32),
            jax.ShapeDtypeStruct((1, 1), jnp.float32),
        ],
        compiler_params=pltpu.CompilerParams(
            dimension_semantics=("arbitrary",),
        ),
    )(x, emb_w)


def _gather_body(emb_hbm, idx_hbm, out_hbm, idx_v, rows_v, sem):
    wid = lax.axis_index("s") * 2 + lax.axis_index("c")
    base = wid * ROWS_PER_W
    for ch in range(NCH):
        off = base + ch * CH
        pltpu.sync_copy(idx_hbm.at[pl.ds(off, CH)], idx_v)
        pltpu.async_copy(emb_hbm.at[idx_v], rows_v, sem).wait()
        pltpu.sync_copy(rows_v, out_hbm.at[pl.ds(off, CH)])


def _gather_call(emb_w, idx_flat):
    call = functools.partial(
        pl.kernel,
        out_type=jax.ShapeDtypeStruct((N, C), jnp.float32),
        mesh=plsc.VectorSubcoreMesh(core_axis_name="c", subcore_axis_name="s",
                                    num_cores=2, num_subcores=16),
        scratch_types=[
            pltpu.VMEM((CH,), jnp.int32),
            pltpu.VMEM((CH, C), jnp.float32),
            pltpu.SemaphoreType.DMA,
        ],
    )(_gather_body)
    return call(emb_w, idx_flat)


def kernel(x, emb_w):
    idx3, loss = _argmin_call(x, emb_w)
    idxs = idx3.reshape(B, L)
    x_q = _gather_call(emb_w, idxs.reshape(N))
    x_q_out = jnp.transpose(x_q.reshape(B, L, C), (0, 2, 1))
    return (x_q_out, idxs, loss.reshape(()))


# speedup vs baseline: 1.7197x; 1.6356x over previous
import functools

import jax
import jax.numpy as jnp
from jax import lax
from jax.experimental import pallas as pl
from jax.experimental.pallas import tpu as pltpu
from jax.experimental.pallas import tpu_sc as plsc

B, C, L = 32, 256, 576
K = 8192
N = B * L
KC = 1024
NKC = K // KC
NW = 32
ROWS_PER_W = N // NW
CH = 96
NCH = ROWS_PER_W // CH
BETA = 0.25


def _argmin_body(x_ref, emb_ref, idx_ref, loss_ref, e2_ref, ee_ref):
    b = pl.program_id(0)

    @pl.when(b == 0)
    def _():
        e = emb_ref[...]
        e2_ref[...] = e + e
        ee_ref[...] = jnp.sum(e * e, axis=1)[None, :]

    xb = x_ref[0]
    xx = jnp.sum(xb * xb, axis=0)

    rmin = None
    rkc = None
    for kc in range(NKC):
        e2 = e2_ref[pl.ds(kc * KC, KC), :]
        ee = ee_ref[0, pl.ds(kc * KC, KC)]
        dots2 = lax.dot_general(
            xb, e2, (((0,), (1,)), ((), ())),
            preferred_element_type=jnp.float32,
        )
        d2 = (xx[:, None] + ee[None, :]) - dots2
        if rmin is None:
            rmin, rkc = d2, jnp.zeros((L, KC), jnp.int32)
        else:
            upd = d2 < rmin
            rmin = jnp.minimum(d2, rmin)
            rkc = jnp.where(upd, kc, rkc)

    minval = jnp.min(rmin, axis=1)
    j = lax.broadcasted_iota(jnp.int32, (L, KC), 1)
    gk = rkc * KC + j
    cand = jnp.where(rmin == minval[:, None], gk, K)
    idx_ref[0, 0, :] = jnp.min(cand, axis=1)

    @pl.when(b == 0)
    def _():
        loss_ref[0, 0] = 0.0

    loss_ref[0, 0] += jnp.sum(jnp.maximum(minval, 0.0))

    @pl.when(b == B - 1)
    def _():
        q = loss_ref[0, 0] / jnp.float32(N * C)
        loss_ref[0, 0] = q + BETA * q


def _argmin_call(x, emb_w):
    return pl.pallas_call(
        _argmin_body,
        grid=(B,),
        in_specs=[
            pl.BlockSpec((1, C, L), lambda b: (b, 0, 0)),
            pl.BlockSpec((K, C), lambda b: (0, 0)),
        ],
        out_specs=[
            pl.BlockSpec((1, 1, L), lambda b: (b, 0, 0)),
            pl.BlockSpec(block_shape=(1, 1), index_map=lambda b: (0, 0),
                         memory_space=pltpu.SMEM),
        ],
        out_shape=[
            jax.ShapeDtypeStruct((B, 1, L), jnp.int32),
            jax.ShapeDtypeStruct((1, 1), jnp.float32),
        ],
        scratch_shapes=[
            pltpu.VMEM((K, C), jnp.float32),
            pltpu.VMEM((1, K), jnp.float32),
        ],
        compiler_params=pltpu.CompilerParams(
            dimension_semantics=("arbitrary",),
        ),
    )(x, emb_w)


def _gather_body(emb_hbm, idx_hbm, out_hbm, idx_v, rows_v, sem):
    wid = lax.axis_index("s") * 2 + lax.axis_index("c")
    base = wid * ROWS_PER_W
    for ch in range(NCH):
        off = base + ch * CH
        pltpu.sync_copy(idx_hbm.at[pl.ds(off, CH)], idx_v)
        pltpu.async_copy(emb_hbm.at[idx_v], rows_v, sem).wait()
        pltpu.sync_copy(rows_v, out_hbm.at[pl.ds(off, CH)])


def _gather_call(emb_w, idx_flat):
    call = functools.partial(
        pl.kernel,
        out_type=jax.ShapeDtypeStruct((N, C), jnp.float32),
        mesh=plsc.VectorSubcoreMesh(core_axis_name="c", subcore_axis_name="s",
                                    num_cores=2, num_subcores=16),
        scratch_types=[
            pltpu.VMEM((CH,), jnp.int32),
            pltpu.VMEM((CH, C), jnp.float32),
            pltpu.SemaphoreType.DMA,
        ],
    )(_gather_body)
    return call(emb_w, idx_flat)


def kernel(x, emb_w):
    idx3, loss = _argmin_call(x, emb_w)
    idxs = idx3.reshape(B, L)
    x_q = _gather_call(emb_w, idxs.reshape(N))
    x_q_out = jnp.transpose(x_q.reshape(B, L, C), (0, 2, 1))
    return (x_q_out, idxs, loss.reshape(()))
